# baseline (speedup 1.0000x reference)
import functools

import jax
import jax.numpy as jnp
from jax import lax
from jax.experimental import pallas as pl
from jax.experimental.pallas import tpu as pltpu
from jax.experimental.pallas import tpu_sc as plsc

B, T, D = 1024, 200, 128
V = 100000

_BBLK = 128
_NBLK_HEAD = 6
_B_HEAD = _NBLK_HEAD * _BBLK
_B_TAIL = B - _B_HEAD
_NBLK_TAIL = _B_TAIL // _BBLK

_NS = 16
_B_PER_W = _B_TAIL // _NS


def _sc_gather_tail(weight, idx):
    mesh = plsc.VectorSubcoreMesh(
        core_axis_name="c", subcore_axis_name="s", num_cores=1
    )

    @functools.partial(
        pl.kernel,
        mesh=mesh,
        out_type=jax.ShapeDtypeStruct((_B_TAIL, D), jnp.float32),
        scratch_types=[
            pltpu.VMEM((_B_PER_W,), jnp.int32),
            pltpu.VMEM((_B_PER_W, D), jnp.float32),
            pltpu.SemaphoreType.DMA,
        ],
    )
    def gather_kernel(table_hbm, idx_hbm, out_hbm, idx_v, rows_v, sem):
        wid = lax.axis_index("s")
        base = wid * _B_PER_W
        pltpu.sync_copy(idx_hbm.at[pl.ds(_B_HEAD + base, _B_PER_W)], idx_v)
        pltpu.async_copy(table_hbm.at[idx_v], rows_v, sem).wait()
        pltpu.sync_copy(rows_v, out_hbm.at[pl.ds(base, _B_PER_W)])

    return gather_kernel(weight, idx)


def _head_body(idx_ref, w_hbm, x_ref, o_ref, rows_v, sem0, sem1):
    i = pl.program_id(0)

    def issue(block, slot, sem):
        for j in range(_BBLK):
            row = idx_ref[block * _BBLK + j]
            pltpu.make_async_copy(
                w_hbm.at[pl.ds(row, 1), :], rows_v.at[slot, pl.ds(j, 1), :], sem
            ).start()

    def drain(block, slot, sem):
        for j in range(_BBLK):
            row = idx_ref[block * _BBLK + j]
            pltpu.make_async_copy(
                w_hbm.at[pl.ds(row, 1), :], rows_v.at[slot, pl.ds(j, 1), :], sem
            ).wait()

    even = lax.rem(i, 2) == 0

    @pl.when(i == 0)
    def _():
        issue(0, 0, sem0)

    @pl.when(jnp.logical_and(i + 1 < _NBLK_HEAD, even))
    def _():
        issue(i + 1, 1, sem1)

    @pl.when(jnp.logical_and(i + 1 < _NBLK_HEAD, jnp.logical_not(even)))
    def _():
        issue(i + 1, 0, sem0)

    @pl.when(even)
    def _():
        drain(i, 0, sem0)

    @pl.when(jnp.logical_not(even))
    def _():
        drain(i, 1, sem1)

    w = jnp.where(even, rows_v[0], rows_v[1])
    s = 2.0 * jax.nn.sigmoid(w)
    o_ref[...] = x_ref[...] * s[:, None, :]


def _tc_head(idx, weight, x):
    grid_spec = pltpu.PrefetchScalarGridSpec(
        num_scalar_prefetch=1,
        grid=(_NBLK_HEAD,),
        in_specs=[
            pl.BlockSpec(memory_space=pltpu.MemorySpace.HBM),
            pl.BlockSpec((_BBLK, T, D), lambda i, idx_ref: (i, 0, 0)),
        ],
        out_specs=pl.BlockSpec((_BBLK, T, D), lambda i, idx_ref: (i, 0, 0)),
        scratch_shapes=[
            pltpu.VMEM((2, _BBLK, D), jnp.float32),
            pltpu.SemaphoreType.DMA,
            pltpu.SemaphoreType.DMA,
        ],
    )
    return pl.pallas_call(
        _head_body,
        grid_spec=grid_spec,
        out_shape=jax.ShapeDtypeStruct((B, T, D), jnp.float32),
    )(idx, weight, x)


def _tail_body(rows_ref, x_ref, part_ref, o_ref):
    i = pl.program_id(0)
    w = rows_ref[pl.ds(i * _BBLK, _BBLK), :]
    s = 2.0 * jax.nn.sigmoid(w)
    o_ref[...] = x_ref[...] * s[:, None, :]


def _tc_tail(rows_tail, x, part):
    return pl.pallas_call(
        _tail_body,
        grid=(_NBLK_TAIL,),
        in_specs=[
            pl.BlockSpec((_B_TAIL, D), lambda i: (0, 0)),
            pl.BlockSpec((_BBLK, T, D), lambda i: (i + _NBLK_HEAD, 0, 0)),
            pl.BlockSpec(memory_space=pltpu.MemorySpace.HBM),
        ],
        out_specs=pl.BlockSpec((_BBLK, T, D), lambda i: (i + _NBLK_HEAD, 0, 0)),
        out_shape=jax.ShapeDtypeStruct((B, T, D), jnp.float32),
        input_output_aliases={2: 0},
    )(rows_tail, x, part)


def kernel(x, spk_id, weight):
    idx = spk_id.reshape(-1)
    rows_tail = _sc_gather_tail(weight, idx)
    part = _tc_head(idx, weight, x)
    return _tc_tail(rows_tail, x, part)


# speedup vs baseline: 1.0073x; 1.0073x over previous
import functools

import jax
import jax.numpy as jnp
from jax import lax
from jax.experimental import pallas as pl
from jax.experimental.pallas import tpu as pltpu
from jax.experimental.pallas import tpu_sc as plsc

B, T, D = 1024, 200, 128
V = 100000

_BBLK = 128
_NBLK_HEAD = 3
_B_HEAD = _NBLK_HEAD * _BBLK
_B_TAIL = B - _B_HEAD
_NBLK_TAIL = _B_TAIL // _BBLK

_NS = 16
_B_PER_W = _B_TAIL // _NS


def _sc_gather_tail(weight, idx):
    mesh = plsc.VectorSubcoreMesh(
        core_axis_name="c", subcore_axis_name="s", num_cores=1
    )

    @functools.partial(
        pl.kernel,
        mesh=mesh,
        out_type=jax.ShapeDtypeStruct((_B_TAIL, D), jnp.float32),
        scratch_types=[
            pltpu.VMEM((_B_PER_W,), jnp.int32),
            pltpu.VMEM((_B_PER_W, D), jnp.float32),
            pltpu.SemaphoreType.DMA,
            pltpu.SemaphoreType.DMA,
        ],
    )
    def gather_kernel(table_hbm, idx_hbm, out_hbm, idx_v, rows_v, sem_g, sem_o):
        wid = lax.axis_index("s")
        base = wid * _B_PER_W
        c0 = 24
        c1 = _B_PER_W - c0
        pltpu.sync_copy(idx_hbm.at[pl.ds(_B_HEAD + base, _B_PER_W)], idx_v)
        g0 = pltpu.async_copy(
            table_hbm.at[idx_v.at[pl.ds(0, c0)]], rows_v.at[pl.ds(0, c0)],
            sem_g,
        )
        g1 = pltpu.async_copy(
            table_hbm.at[idx_v.at[pl.ds(c0, c1)]],
            rows_v.at[pl.ds(c0, c1)], sem_g,
        )
        g0.wait()
        o0 = pltpu.async_copy(
            rows_v.at[pl.ds(0, c0)], out_hbm.at[pl.ds(base, c0)], sem_o
        )
        g1.wait()
        pltpu.sync_copy(
            rows_v.at[pl.ds(c0, c1)], out_hbm.at[pl.ds(base + c0, c1)]
        )
        o0.wait()

    return gather_kernel(weight, idx)


def _head_body(idx_ref, w_hbm, x_ref, o_ref, rows_v, sem0, sem1):
    i = pl.program_id(0)

    def issue(block, slot, sem):
        for j in range(_BBLK):
            row = idx_ref[block * _BBLK + j]
            pltpu.make_async_copy(
                w_hbm.at[pl.ds(row, 1), :], rows_v.at[slot, pl.ds(j, 1), :], sem
            ).start()

    def drain(block, slot, sem):
        for j in range(_BBLK):
            row = idx_ref[block * _BBLK + j]
            pltpu.make_async_copy(
                w_hbm.at[pl.ds(row, 1), :], rows_v.at[slot, pl.ds(j, 1), :], sem
            ).wait()

    even = lax.rem(i, 2) == 0

    @pl.when(i == 0)
    def _():
        issue(0, 0, sem0)

    @pl.when(jnp.logical_and(i + 1 < _NBLK_HEAD, even))
    def _():
        issue(i + 1, 1, sem1)

    @pl.when(jnp.logical_and(i + 1 < _NBLK_HEAD, jnp.logical_not(even)))
    def _():
        issue(i + 1, 0, sem0)

    @pl.when(even)
    def _():
        drain(i, 0, sem0)

    @pl.when(jnp.logical_not(even))
    def _():
        drain(i, 1, sem1)

    w = jnp.where(even, rows_v[0], rows_v[1])
    s = 2.0 * jax.nn.sigmoid(w)
    o_ref[...] = x_ref[...] * s[:, None, :]


def _tc_head(idx, weight, x):
    grid_spec = pltpu.PrefetchScalarGridSpec(
        num_scalar_prefetch=1,
        grid=(_NBLK_HEAD,),
        in_specs=[
            pl.BlockSpec(memory_space=pltpu.MemorySpace.HBM),
            pl.BlockSpec((_BBLK, T, D), lambda i, idx_ref: (i, 0, 0)),
        ],
        out_specs=pl.BlockSpec((_BBLK, T, D), lambda i, idx_ref: (i, 0, 0)),
        scratch_shapes=[
            pltpu.VMEM((2, _BBLK, D), jnp.float32),
            pltpu.SemaphoreType.DMA,
            pltpu.SemaphoreType.DMA,
        ],
    )
    return pl.pallas_call(
        _head_body,
        grid_spec=grid_spec,
        out_shape=jax.ShapeDtypeStruct((B, T, D), jnp.float32),
    )(idx, weight, x)


def _tail_body(rows_ref, x_ref, part_ref, o_ref):
    i = pl.program_id(0)
    w = rows_ref[pl.ds(i * _BBLK, _BBLK), :]
    s = 2.0 * jax.nn.sigmoid(w)
    o_ref[...] = x_ref[...] * s[:, None, :]


def _tc_tail(rows_tail, x, part):
    return pl.pallas_call(
        _tail_body,
        grid=(_NBLK_TAIL,),
        in_specs=[
            pl.BlockSpec((_B_TAIL, D), lambda i: (0, 0)),
            pl.BlockSpec((_BBLK, T, D), lambda i: (i + _NBLK_HEAD, 0, 0)),
            pl.BlockSpec(memory_space=pltpu.MemorySpace.HBM),
        ],
        out_specs=pl.BlockSpec((_BBLK, T, D), lambda i: (i + _NBLK_HEAD, 0, 0)),
        out_shape=jax.ShapeDtypeStruct((B, T, D), jnp.float32),
        input_output_aliases={2: 0},
    )(rows_tail, x, part)


def kernel(x, spk_id, weight):
    idx = spk_id.reshape(-1)
    rows_tail = _sc_gather_tail(weight, idx)
    part = _tc_head(idx, weight, x)
    return _tc_tail(rows_tail, x, part)
